# baseline (speedup 1.0000x reference)
import functools

import jax
import jax.numpy as jnp
from jax import lax
from jax.experimental import pallas as pl
from jax.experimental.pallas import tpu as pltpu
from jax.experimental.pallas import tpu_sc as plsc

_N = 10000
_E = 320000
_NOUT = 32
_NC = 2
_NS = 16
_NW = _NC * _NS
_EPW = _E // _NW
_B = 125
_C = _EPW // _B
_NBUF = 8


def _spmm_sc(table, src_w, dst_w, zeros):
    n, f = table.shape
    nw, c, b = src_w.shape
    rp = 624
    tail_start = rp * _NS
    tail = n - tail_start

    mesh = plsc.VectorSubcoreMesh(
        core_axis_name="core", subcore_axis_name="subcore",
        num_cores=_NC, num_subcores=_NS)

    @functools.partial(
        pl.kernel,
        out_type=jax.ShapeDtypeStruct((_NC, n, f), jnp.float32),
        mesh=mesh,
        compiler_params=pltpu.CompilerParams(use_tc_tiling_on_sc=False),
        scratch_types=[
            pltpu.VMEM((c, b), jnp.int32),
            pltpu.VMEM((c, b), jnp.int32),
            [pltpu.VMEM((b, f), jnp.float32) for _ in range(_NBUF)],
            pltpu.VMEM_SHARED((n, f), jnp.float32),
            [pltpu.SemaphoreType.DMA for _ in range(_NBUF)],
            [pltpu.SemaphoreType.DMA for _ in range(_NBUF)],
        ],
    )
    def k(table_hbm, src_hbm, dst_hbm, zeros_hbm, out_hbm,
          src_v, dst_v, rows, acc_s, gsem, ssem):
        cid = lax.axis_index("core")
        sid = lax.axis_index("subcore")
        wid = cid * _NS + sid
        pltpu.sync_copy(src_hbm.at[wid], src_v)
        pltpu.sync_copy(dst_hbm.at[wid], dst_v)
        sl = pl.ds(sid * rp, rp)
        pltpu.sync_copy(zeros_hbm.at[sl], acc_s.at[sl])

        @pl.when(sid == _NS - 1)
        def _():
            tsl = pl.ds(tail_start, tail)
            pltpu.sync_copy(zeros_hbm.at[tsl], acc_s.at[tsl])

        plsc.subcore_barrier()

        def _gather(j, t):
            pltpu.async_copy(table_hbm.at[src_v.at[j]], rows[t], gsem[t])

        def _scatter(j, t):
            pltpu.async_copy(rows[t], acc_s.at[dst_v.at[j]], ssem[t],
                             add=True)

        for t in range(_NBUF):
            _gather(t, t)

        @pl.loop(0, c // _NBUF)
        def _(i):
            base = i * _NBUF
            for t in range(_NBUF):
                j = base + t
                ta = (t + 4) % _NBUF

                @pl.when((j >= 4) & (j < c - 4))
                def _():
                    pltpu.make_async_copy(
                        rows[ta], acc_s.at[dst_v.at[j]], ssem[ta]).wait()
                    _gather(j + 4, ta)

                pltpu.make_async_copy(
                    table_hbm.at[src_v.at[j]], rows[t], gsem[t]).wait()
                _scatter(j, t)

        for t in range(_NBUF):
            pltpu.make_async_copy(
                rows[t], acc_s.at[dst_v.at[0]], ssem[t]).wait()

        plsc.subcore_barrier()
        pltpu.sync_copy(acc_s.at[sl], out_hbm.at[cid].at[sl])

        @pl.when(sid == _NS - 1)
        def _():
            tsl = pl.ds(tail_start, tail)
            pltpu.sync_copy(acc_s.at[tsl], out_hbm.at[cid].at[tsl])

    return k(table, src_w, dst_w, zeros)


def _spmm3_sc(table, w2, src_w, dst_w, zeros):
    n, f = table.shape
    nw, c, b = src_w.shape
    rp = 624
    tail_start = rp * _NS
    tail = n - tail_start
    rpw = n // _NW

    mesh = plsc.VectorSubcoreMesh(
        core_axis_name="core", subcore_axis_name="subcore",
        num_cores=_NC, num_subcores=_NS)
    sem_mesh = plsc.VectorSubcoreMesh(
        core_axis_name="xc", subcore_axis_name="xs",
        num_cores=_NC, num_subcores=_NS)

    @functools.partial(
        pl.kernel,
        out_type=(
            jax.ShapeDtypeStruct((_NC, n, f), jnp.float32),
            jax.ShapeDtypeStruct((_NC, n, f), jnp.float32),
            jax.ShapeDtypeStruct((n, f), jnp.float32),
        ),
        mesh=mesh,
        compiler_params=pltpu.CompilerParams(use_tc_tiling_on_sc=False),
        scratch_types=[
            pltpu.VMEM((c, b), jnp.int32),
            pltpu.VMEM((c, b), jnp.int32),
            [pltpu.VMEM((b, f), jnp.float32) for _ in range(_NBUF)],
            pltpu.VMEM((rpw + 16, f), jnp.float32),
            pltpu.VMEM((rpw + 16, f), jnp.float32),
            pltpu.VMEM((rpw + 16, f), jnp.float32),
            pltpu.VMEM((f, f), jnp.float32),
            pltpu.VMEM_SHARED((n, f), jnp.float32),
            [pltpu.SemaphoreType.DMA for _ in range(_NBUF)],
            [pltpu.SemaphoreType.DMA for _ in range(_NBUF)],
            pltpu.SemaphoreType.REGULAR @ sem_mesh,
        ],
    )
    def k(table_hbm, w2_hbm, src_hbm, dst_hbm, zeros_hbm,
          out_hbm, pmid_hbm, ts_hbm,
          src_v, dst_v, rows, ablk, pblk, tblk, w2_v, acc_s,
          gsem, ssem, xsem):
        cid = lax.axis_index("core")
        sid = lax.axis_index("subcore")
        wid = cid * _NS + sid
        pltpu.sync_copy(src_hbm.at[wid], src_v)
        pltpu.sync_copy(dst_hbm.at[wid], dst_v)
        sl = pl.ds(sid * rp, rp)
        tsl = pl.ds(tail_start, tail)
        pltpu.sync_copy(zeros_hbm.at[sl], acc_s.at[sl])

        @pl.when(sid == _NS - 1)
        def _():
            pltpu.sync_copy(zeros_hbm.at[tsl], acc_s.at[tsl])

        plsc.subcore_barrier()

        def _gather(tbl, j, t):
            pltpu.async_copy(tbl.at[src_v.at[j]], rows[t], gsem[t])

        def _ring(tbl):
            for t in range(_NBUF):
                _gather(tbl, t, t)

            @pl.loop(0, c // _NBUF)
            def _(i):
                base = i * _NBUF
                for t in range(_NBUF):
                    j = base + t
                    ta = (t + 4) % _NBUF

                    @pl.when((j >= 4) & (j < c - 4))
                    def _():
                        pltpu.make_async_copy(
                            rows[ta], acc_s.at[dst_v.at[j]], ssem[ta]).wait()
                        _gather(tbl, j + 4, ta)

                    pltpu.make_async_copy(
                        tbl.at[src_v.at[j]], rows[t], gsem[t]).wait()
                    pltpu.async_copy(rows[t], acc_s.at[dst_v.at[j]], ssem[t],
                                     add=True)

            for t in range(_NBUF):
                pltpu.make_async_copy(
                    rows[t], acc_s.at[dst_v.at[0]], ssem[t]).wait()

        def _xbarrier():
            plsc.subcore_barrier()
            pltpu.semaphore_signal(xsem, 1, device_id={"xc": 1 - cid, "xs": sid})
            pltpu.semaphore_wait(xsem, 1)

        pltpu.sync_copy(w2_hbm, w2_v)

        blk = pl.ds(wid * rpw, rpw)
        nr = jnp.where(wid == _NW - 1, rpw + tail, rpw)

        def _mid_phase(with_mm):
            pltpu.sync_copy(acc_s.at[sl], pmid_hbm.at[cid].at[sl])

            @pl.when(sid == _NS - 1)
            def _():
                pltpu.sync_copy(acc_s.at[tsl], pmid_hbm.at[cid].at[tsl])

            pltpu.sync_copy(acc_s.at[blk], ablk.at[pl.ds(0, rpw)])

            @pl.when(wid == _NW - 1)
            def _():
                pltpu.sync_copy(acc_s.at[tsl], ablk.at[pl.ds(rpw, tail)])

            plsc.subcore_barrier()
            pltpu.semaphore_signal(
                xsem, 1, device_id={"xc": 1 - cid, "xs": sid})
            pltpu.sync_copy(zeros_hbm.at[sl], acc_s.at[sl])

            @pl.when(sid == _NS - 1)
            def _():
                pltpu.sync_copy(zeros_hbm.at[tsl], acc_s.at[tsl])

            pltpu.semaphore_wait(xsem, 1)

            pltpu.sync_copy(pmid_hbm.at[1 - cid].at[blk],
                            pblk.at[pl.ds(0, rpw)])

            @pl.when(wid == _NW - 1)
            def _():
                pltpu.sync_copy(pmid_hbm.at[1 - cid].at[tsl],
                                pblk.at[pl.ds(rpw, tail)])

            dst_blk = ablk if with_mm else tblk

            @pl.loop(0, nr)
            def _(r):
                for g in range(f // 16):
                    gsl = pl.ds(g * 16, 16)
                    v = ablk[r, gsl] + pblk[r, gsl]
                    dst_blk[r, gsl] = jnp.where(
                        v > 0, v, jnp.exp(jnp.minimum(v, 0.0)) - 1.0)

            if with_mm:
                @pl.loop(0, nr // 4)
                def _(i):
                    base = i * 4
                    hvs = [[ablk[base + r, pl.ds(kc * 16, 16)]
                            for kc in range(f // 16)] for r in range(4)]
                    for g in range(f // 16):
                        gsl = pl.ds(g * 16, 16)
                        acc = [None] * 4
                        for kk in range(f):
                            wv = w2_v[kk, gsl]
                            for r in range(4):
                                term = hvs[r][kk // 16][kk % 16] * wv
                                acc[r] = term if acc[r] is None \
                                    else acc[r] + term
                        for r in range(4):
                            tblk[base + r, gsl] = acc[r]

            pltpu.sync_copy(tblk.at[pl.ds(0, rpw)], ts_hbm.at[blk])

            @pl.when(wid == _NW - 1)
            def _():
                pltpu.sync_copy(tblk.at[pl.ds(rpw, tail)], ts_hbm.at[tsl])

            _xbarrier()

        _ring(table_hbm)
        plsc.subcore_barrier()
        _mid_phase(with_mm=True)

        _ring(ts_hbm)
        plsc.subcore_barrier()
        _mid_phase(with_mm=False)

        _ring(ts_hbm)
        plsc.subcore_barrier()
        pltpu.sync_copy(acc_s.at[sl], out_hbm.at[cid].at[sl])

        @pl.when(sid == _NS - 1)
        def _():
            pltpu.sync_copy(acc_s.at[tsl], out_hbm.at[cid].at[tsl])

    return k(table, w2, src_w, dst_w, zeros)


def _dot(a, b):
    return jnp.dot(a, b, preferred_element_type=jnp.float32,
                   precision=lax.Precision.HIGHEST)


def _mm0(x, w):
    def body(x_ref, w_ref, o_ref):
        o_ref[...] = _dot(x_ref[...], w_ref[...])
    return pl.pallas_call(
        body,
        out_shape=jax.ShapeDtypeStruct((x.shape[0], w.shape[1]), jnp.float32),
    )(x, w)


def _act(h, act):
    if act == "relu":
        return jnp.maximum(h, 0.0)
    if act == "elu":
        return jnp.where(h > 0, h, jnp.exp(jnp.minimum(h, 0.0)) - 1.0)
    return h


def _mm_fused(p, w, act):
    def body(p_ref, w_ref, o_ref):
        o_ref[...] = _dot(_act(p_ref[0] + p_ref[1], act), w_ref[...])
    return pl.pallas_call(
        body,
        out_shape=jax.ShapeDtypeStruct((p.shape[1], w.shape[1]), jnp.float32),
    )(p, w)


def _act_fused(p, act):
    def body(p_ref, o_ref):
        o_ref[...] = _act(p_ref[0] + p_ref[1], act)
    return pl.pallas_call(
        body,
        out_shape=jax.ShapeDtypeStruct(p.shape[1:], jnp.float32),
    )(p)


def kernel(x, edge_index, W0, W1, W2, W3):
    src_w = edge_index[0].reshape(_NW, _C, _B)
    dst_w = edge_index[1].reshape(_NW, _C, _B)
    z64 = jnp.zeros((_N, 64), jnp.float32)
    z32 = jnp.zeros((_N, 32), jnp.float32)

    t0 = _mm0(x, W0)
    p0 = _spmm_sc(t0, src_w, dst_w, z64)
    t1 = _mm_fused(p0, W1, "relu")
    p3, _, _ = _spmm3_sc(t1, W2, src_w, dst_w, z32)
    out = _mm_fused(p3, W3, None)
    return (out[:, :_NOUT], out[:, _NOUT:])


# speedup vs baseline: 1.0247x; 1.0247x over previous
import functools

import jax
import jax.numpy as jnp
from jax import lax
from jax.experimental import pallas as pl
from jax.experimental.pallas import tpu as pltpu
from jax.experimental.pallas import tpu_sc as plsc

_N = 10000
_E = 320000
_NOUT = 32
_NC = 2
_NS = 16
_NW = _NC * _NS
_EPW = _E // _NW
_B = 125
_C = _EPW // _B
_NBUF = 8


def _spmm_sc(table, src_w, dst_w, zeros):
    n, f = table.shape
    nw, c, b = src_w.shape
    rp = 624
    tail_start = rp * _NS
    tail = n - tail_start

    mesh = plsc.VectorSubcoreMesh(
        core_axis_name="core", subcore_axis_name="subcore",
        num_cores=_NC, num_subcores=_NS)

    @functools.partial(
        pl.kernel,
        out_type=jax.ShapeDtypeStruct((_NC, n, f), jnp.float32),
        mesh=mesh,
        compiler_params=pltpu.CompilerParams(use_tc_tiling_on_sc=False),
        scratch_types=[
            pltpu.VMEM((c, b), jnp.int32),
            pltpu.VMEM((c, b), jnp.int32),
            [pltpu.VMEM((b, f), jnp.float32) for _ in range(_NBUF)],
            pltpu.VMEM_SHARED((n, f), jnp.float32),
            [pltpu.SemaphoreType.DMA for _ in range(_NBUF)],
            [pltpu.SemaphoreType.DMA for _ in range(_NBUF)],
        ],
    )
    def k(table_hbm, src_hbm, dst_hbm, zeros_hbm, out_hbm,
          src_v, dst_v, rows, acc_s, gsem, ssem):
        cid = lax.axis_index("core")
        sid = lax.axis_index("subcore")
        wid = cid * _NS + sid
        pltpu.sync_copy(src_hbm.at[wid], src_v)
        pltpu.sync_copy(dst_hbm.at[wid], dst_v)
        sl = pl.ds(sid * rp, rp)
        pltpu.sync_copy(zeros_hbm.at[sl], acc_s.at[sl])

        @pl.when(sid == _NS - 1)
        def _():
            tsl = pl.ds(tail_start, tail)
            pltpu.sync_copy(zeros_hbm.at[tsl], acc_s.at[tsl])

        plsc.subcore_barrier()

        def _gather(j, t):
            pltpu.async_copy(table_hbm.at[src_v.at[j]], rows[t], gsem[t])

        def _scatter(j, t):
            pltpu.async_copy(rows[t], acc_s.at[dst_v.at[j]], ssem[t],
                             add=True)

        for t in range(_NBUF):
            _gather(t, t)

        @pl.loop(0, c // _NBUF)
        def _(i):
            base = i * _NBUF
            for t in range(_NBUF):
                j = base + t
                ta = (t + 4) % _NBUF

                @pl.when((j >= 4) & (j < c - 4))
                def _():
                    pltpu.make_async_copy(
                        rows[ta], acc_s.at[dst_v.at[j]], ssem[ta]).wait()
                    _gather(j + 4, ta)

                pltpu.make_async_copy(
                    table_hbm.at[src_v.at[j]], rows[t], gsem[t]).wait()
                _scatter(j, t)

        for t in range(_NBUF):
            pltpu.make_async_copy(
                rows[t], acc_s.at[dst_v.at[0]], ssem[t]).wait()

        plsc.subcore_barrier()
        pltpu.sync_copy(acc_s.at[sl], out_hbm.at[cid].at[sl])

        @pl.when(sid == _NS - 1)
        def _():
            tsl = pl.ds(tail_start, tail)
            pltpu.sync_copy(acc_s.at[tsl], out_hbm.at[cid].at[tsl])

    return k(table, src_w, dst_w, zeros)


def _spmm3_sc(table, w2, src_w, dst_w, zeros):
    n, f = table.shape
    nw, c, b = src_w.shape
    rp = 624
    tail_start = rp * _NS
    tail = n - tail_start
    rpw = n // _NW

    mesh = plsc.VectorSubcoreMesh(
        core_axis_name="core", subcore_axis_name="subcore",
        num_cores=_NC, num_subcores=_NS)
    sem_mesh = plsc.VectorSubcoreMesh(
        core_axis_name="xc", subcore_axis_name="xs",
        num_cores=_NC, num_subcores=_NS)

    @functools.partial(
        pl.kernel,
        out_type=(
            jax.ShapeDtypeStruct((_NC, n, f), jnp.float32),
            jax.ShapeDtypeStruct((_NC, n, f), jnp.float32),
            jax.ShapeDtypeStruct((n, f), jnp.float32),
        ),
        mesh=mesh,
        compiler_params=pltpu.CompilerParams(use_tc_tiling_on_sc=False),
        scratch_types=[
            pltpu.VMEM((c, b), jnp.int32),
            pltpu.VMEM((c, b), jnp.int32),
            [pltpu.VMEM((b, f), jnp.float32) for _ in range(_NBUF)],
            pltpu.VMEM((rpw + 16, f), jnp.float32),
            pltpu.VMEM((rpw + 16, f), jnp.float32),
            pltpu.VMEM((rpw + 16, f), jnp.float32),
            pltpu.VMEM((f, f), jnp.float32),
            pltpu.VMEM_SHARED((n, f), jnp.float32),
            [pltpu.SemaphoreType.DMA for _ in range(_NBUF)],
            [pltpu.SemaphoreType.DMA for _ in range(_NBUF)],
            pltpu.SemaphoreType.REGULAR @ sem_mesh,
        ],
    )
    def k(table_hbm, w2_hbm, src_hbm, dst_hbm, zeros_hbm,
          out_hbm, pmid_hbm, ts_hbm,
          src_v, dst_v, rows, ablk, pblk, tblk, w2_v, acc_s,
          gsem, ssem, xsem):
        cid = lax.axis_index("core")
        sid = lax.axis_index("subcore")
        wid = cid * _NS + sid
        pltpu.sync_copy(src_hbm.at[wid], src_v)
        pltpu.sync_copy(dst_hbm.at[wid], dst_v)
        sl = pl.ds(sid * rp, rp)
        tsl = pl.ds(tail_start, tail)
        pltpu.sync_copy(zeros_hbm.at[sl], acc_s.at[sl])

        @pl.when(sid == _NS - 1)
        def _():
            pltpu.sync_copy(zeros_hbm.at[tsl], acc_s.at[tsl])

        plsc.subcore_barrier()

        def _gather(tbl, j, t):
            pltpu.async_copy(tbl.at[src_v.at[j]], rows[t], gsem[t])

        def _ring(tbl):
            for t in range(_NBUF):
                _gather(tbl, t, t)

            @pl.loop(0, c // _NBUF)
            def _(i):
                base = i * _NBUF
                for t in range(_NBUF):
                    j = base + t
                    ta = (t + 4) % _NBUF

                    @pl.when((j >= 4) & (j < c - 4))
                    def _():
                        pltpu.make_async_copy(
                            rows[ta], acc_s.at[dst_v.at[j]], ssem[ta]).wait()
                        _gather(tbl, j + 4, ta)

                    pltpu.make_async_copy(
                        tbl.at[src_v.at[j]], rows[t], gsem[t]).wait()
                    pltpu.async_copy(rows[t], acc_s.at[dst_v.at[j]], ssem[t],
                                     add=True)

            for t in range(_NBUF):
                pltpu.make_async_copy(
                    rows[t], acc_s.at[dst_v.at[0]], ssem[t]).wait()

        def _xbarrier():
            plsc.subcore_barrier()
            pltpu.semaphore_signal(xsem, 1, device_id={"xc": 1 - cid, "xs": sid})
            pltpu.semaphore_wait(xsem, 1)

        pltpu.sync_copy(w2_hbm, w2_v)

        blk = pl.ds(wid * rpw, rpw)
        nr = jnp.where(wid == _NW - 1, rpw + tail, rpw)

        def _mid_phase(with_mm):
            pltpu.sync_copy(acc_s.at[sl], pmid_hbm.at[cid].at[sl])

            @pl.when(sid == _NS - 1)
            def _():
                pltpu.sync_copy(acc_s.at[tsl], pmid_hbm.at[cid].at[tsl])

            pltpu.sync_copy(acc_s.at[blk], ablk.at[pl.ds(0, rpw)])

            @pl.when(wid == _NW - 1)
            def _():
                pltpu.sync_copy(acc_s.at[tsl], ablk.at[pl.ds(rpw, tail)])

            plsc.subcore_barrier()
            pltpu.semaphore_signal(
                xsem, 1, device_id={"xc": 1 - cid, "xs": sid})
            pltpu.sync_copy(zeros_hbm.at[sl], acc_s.at[sl])

            @pl.when(sid == _NS - 1)
            def _():
                pltpu.sync_copy(zeros_hbm.at[tsl], acc_s.at[tsl])

            pltpu.semaphore_wait(xsem, 1)

            pltpu.sync_copy(pmid_hbm.at[1 - cid].at[blk],
                            pblk.at[pl.ds(0, rpw)])

            @pl.when(wid == _NW - 1)
            def _():
                pltpu.sync_copy(pmid_hbm.at[1 - cid].at[tsl],
                                pblk.at[pl.ds(rpw, tail)])

            dst_blk = ablk if with_mm else tblk

            @pl.loop(0, nr)
            def _(r):
                for g in range(f // 16):
                    gsl = pl.ds(g * 16, 16)
                    v = ablk[r, gsl] + pblk[r, gsl]
                    dst_blk[r, gsl] = jnp.where(
                        v > 0, v, jnp.exp(jnp.minimum(v, 0.0)) - 1.0)

            if with_mm:
                ng = f // 16

                @pl.loop(0, nr // 4)
                def _(i):
                    base = i * 4
                    hvs = [[ablk[base + r, pl.ds(kc * 16, 16)]
                            for kc in range(ng)] for r in range(4)]
                    acc = [[None] * ng for _ in range(4)]
                    for kk in range(f):
                        wvs = [w2_v[kk, pl.ds(g * 16, 16)] for g in range(ng)]
                        for r in range(4):
                            hb = hvs[r][kk // 16][kk % 16]
                            for g in range(ng):
                                term = hb * wvs[g]
                                acc[r][g] = term if acc[r][g] is None \
                                    else acc[r][g] + term
                    for r in range(4):
                        for g in range(ng):
                            tblk[base + r, pl.ds(g * 16, 16)] = acc[r][g]

            pltpu.sync_copy(tblk.at[pl.ds(0, rpw)], ts_hbm.at[blk])

            @pl.when(wid == _NW - 1)
            def _():
                pltpu.sync_copy(tblk.at[pl.ds(rpw, tail)], ts_hbm.at[tsl])

            _xbarrier()

        _ring(table_hbm)
        plsc.subcore_barrier()
        _mid_phase(with_mm=True)

        _ring(ts_hbm)
        plsc.subcore_barrier()
        _mid_phase(with_mm=False)

        _ring(ts_hbm)
        plsc.subcore_barrier()
        pltpu.sync_copy(acc_s.at[sl], out_hbm.at[cid].at[sl])

        @pl.when(sid == _NS - 1)
        def _():
            pltpu.sync_copy(acc_s.at[tsl], out_hbm.at[cid].at[tsl])

    return k(table, w2, src_w, dst_w, zeros)


def _dot(a, b):
    return jnp.dot(a, b, preferred_element_type=jnp.float32,
                   precision=lax.Precision.HIGHEST)


def _mm0(x, w):
    def body(x_ref, w_ref, o_ref):
        o_ref[...] = _dot(x_ref[...], w_ref[...])
    return pl.pallas_call(
        body,
        out_shape=jax.ShapeDtypeStruct((x.shape[0], w.shape[1]), jnp.float32),
    )(x, w)


def _act(h, act):
    if act == "relu":
        return jnp.maximum(h, 0.0)
    if act == "elu":
        return jnp.where(h > 0, h, jnp.exp(jnp.minimum(h, 0.0)) - 1.0)
    return h


def _mm_fused(p, w, act):
    def body(p_ref, w_ref, o_ref):
        o_ref[...] = _dot(_act(p_ref[0] + p_ref[1], act), w_ref[...])
    return pl.pallas_call(
        body,
        out_shape=jax.ShapeDtypeStruct((p.shape[1], w.shape[1]), jnp.float32),
    )(p, w)


def _act_fused(p, act):
    def body(p_ref, o_ref):
        o_ref[...] = _act(p_ref[0] + p_ref[1], act)
    return pl.pallas_call(
        body,
        out_shape=jax.ShapeDtypeStruct(p.shape[1:], jnp.float32),
    )(p)


def kernel(x, edge_index, W0, W1, W2, W3):
    src_w = edge_index[0].reshape(_NW, _C, _B)
    dst_w = edge_index[1].reshape(_NW, _C, _B)
    z64 = jnp.zeros((_N, 64), jnp.float32)
    z32 = jnp.zeros((_N, 32), jnp.float32)

    t0 = _mm0(x, W0)
    p0 = _spmm_sc(t0, src_w, dst_w, z64)
    t1 = _mm_fused(p0, W1, "relu")
    p3, _, _ = _spmm3_sc(t1, W2, src_w, dst_w, z32)
    out = _mm_fused(p3, W3, None)
    return (out[:, :_NOUT], out[:, _NOUT:])
